# Initial kernel scaffold; baseline (speedup 1.0000x reference)
import jax
import jax.numpy as jnp
from jax.experimental import pallas as pl


def kernel(species, edge_index, edge_attr, edge_vec, W_sp, b_sp, W_e, b_e, W_msg, b_msg, W_vec, W_dh, b_dh, w_dv):
    raise NotImplementedError("write your pallas kernel here")


# speedup vs baseline: 86.7562x; 86.7562x over previous
import functools

import jax
import jax.numpy as jnp
from jax import lax
from jax.experimental import pallas as pl
from jax.experimental.pallas import tpu as pltpu
from jax.experimental.pallas import tpu_sc as plsc

N_NODES = 10000
N_EDGES = 160000
CUTOFF = 5.0

NC = 2
NS = 16
NW = NC * NS
NPAD = 10240
NH = NPAD // 2
EPAD = 163840
EPT = EPAD // NW
CH = 512
NCHUNK = EPT // CH
GRP = CH // 16

_PI = 3.14159265358979323846
_C3 = -1.0 / 6.0
_C5 = 1.0 / 120.0
_C7 = -1.0 / 5040.0
_C9 = 1.0 / 362880.0


def _prep_body(sp_ref, m_ref, c_ref, mb_ref, cb_ref, table_ref, base_ref):
    sp = sp_ref[...]
    table = jax.lax.dot_general(
        sp, m_ref[...], (((1,), (0,)), ((), ())),
        preferred_element_type=jnp.float32,
        precision=jax.lax.Precision.HIGHEST) + c_ref[...]
    base = jax.lax.dot_general(
        sp, mb_ref[...], (((1,), (0,)), ((), ())),
        preferred_element_type=jnp.float32,
        precision=jax.lax.Precision.HIGHEST) + cb_ref[...]
    rows10 = lax.broadcasted_iota(jnp.int32, (NPAD, 10), 0)
    table_ref[...] = jnp.where(rows10 < N_NODES, table, 0.0)
    rows4 = lax.broadcasted_iota(jnp.int32, (NPAD, 4), 0)
    cols4 = lax.broadcasted_iota(jnp.int32, (NPAD, 4), 1)
    base4 = jnp.broadcast_to(base, (NPAD, 4))
    base_ref[...] = jnp.where((rows4 < N_NODES) & (cols4 == 0), base4, 0.0)


def _prep_call(sp8, m8, c10, mb8, cb):
    return pl.pallas_call(
        _prep_body,
        out_shape=(
            jax.ShapeDtypeStruct((NPAD, 10), jnp.float32),
            jax.ShapeDtypeStruct((NPAD, 4), jnp.float32),
        ),
    )(sp8, m8, c10, mb8, cb)


def _sc_body(table_hbm, src_hbm, dst_hbm, ea_hbm, out_hbm,
             table_v, src_v, dst_v, ea_v, acc_v):
    cid = lax.axis_index("c")
    sid = lax.axis_index("s")
    wid = cid * NS + sid

    pltpu.sync_copy(table_hbm, table_v)
    lane = lax.iota(jnp.int32, 16)
    zeros16 = jnp.zeros((16,), jnp.float32)
    e0 = wid * EPT

    for half in range(2):
        rbase = half * NH

        def zero(i, carry):
            acc_v[pl.ds(i * 16, 16)] = zeros16
            return carry

        lax.fori_loop(0, (NH * 4) // 16, zero, 0, unroll=4)

        for k in range(NCHUNK):
            off = pl.multiple_of(e0 + k * CH, CH)
            pltpu.sync_copy(src_hbm.at[pl.ds(off, CH)], src_v)
            pltpu.sync_copy(dst_hbm.at[pl.ds(off, CH)], dst_v)
            for j in range(4):
                pltpu.sync_copy(ea_hbm.at[j, pl.ds(off, CH)], ea_v.at[j])

            def group(g, carry):
                s16 = src_v[pl.ds(g * 16, 16)] * 10
                tv = [plsc.load_gather(table_v, [s16 + j])
                      for j in range(10)]
                ea0 = ea_v[0, pl.ds(g * 16, 16)]
                ea1 = ea_v[1, pl.ds(g * 16, 16)]
                ea2 = ea_v[2, pl.ds(g * 16, 16)]
                ln = ea_v[3, pl.ds(g * 16, 16)]
                x = jnp.minimum(ln * (1.0 / CUTOFF), 1.0)
                z = x * _PI - (0.5 * _PI)
                z2 = z * z
                poly = _C3 + z2 * (_C5 + z2 * (_C7 + z2 * _C9))
                env = 0.5 - 0.5 * (z + z * z2 * poly)
                inv = 1.0 / (ln + 1e-8)
                dotp = ea0 * tv[0] + ea1 * tv[1] + ea2 * tv[2] \
                    + ln * tv[3] + tv[4]
                dotq = ea0 * tv[5] + ea1 * tv[6] + ea2 * tv[7] \
                    + ln * tv[8] + tv[9]
                t = env * dotp
                sv = env * dotq * inv
                d16 = dst_v[pl.ds(g * 16, 16)] - rbase
                m = (d16 >= 0) & (d16 < NH)
                ix = jnp.where(m, d16, 0) * 4
                plsc.addupdate_scatter(acc_v, [ix], t, mask=m)
                plsc.addupdate_scatter(acc_v, [ix + 1], sv * ea0, mask=m)
                plsc.addupdate_scatter(acc_v, [ix + 2], sv * ea1, mask=m)
                plsc.addupdate_scatter(acc_v, [ix + 3], sv * ea2, mask=m)
                return carry

            lax.fori_loop(0, GRP, group, 0, unroll=1)

        rb = pl.multiple_of(rbase * 4, NH * 4)
        pltpu.sync_copy(acc_v, out_hbm.at[wid, pl.ds(rb, NH * 4)])


def _sc_call(table, src_p, dst_p, ea_p):
    mesh = plsc.VectorSubcoreMesh(core_axis_name="c", subcore_axis_name="s",
                                  num_cores=NC, num_subcores=NS)
    f = functools.partial(
        pl.kernel,
        out_type=jax.ShapeDtypeStruct((NW, NPAD * 4), jnp.float32),
        mesh=mesh,
        compiler_params=pltpu.CompilerParams(needs_layout_passes=False,
                                             use_tc_tiling_on_sc=False),
        scratch_types=[
            pltpu.VMEM((NPAD * 10,), jnp.float32),
            pltpu.VMEM((CH,), jnp.int32),
            pltpu.VMEM((CH,), jnp.int32),
            pltpu.VMEM((4, CH), jnp.float32),
            pltpu.VMEM((NH * 4,), jnp.float32),
        ],
    )(_sc_body)
    return f(table, src_p, dst_p, ea_p)


def _reduce_body(part_ref, base_ref, out_ref):
    out_ref[...] = jnp.sum(part_ref[...], axis=0, keepdims=True) + base_ref[...]


def _reduce_call(part2d, base2d):
    return pl.pallas_call(
        _reduce_body,
        out_shape=jax.ShapeDtypeStruct((1, NPAD * 4), jnp.float32),
    )(part2d, base2d)


def _emit_body(comb_ref, oh_ref, ov_ref):
    p = comb_ref[...]
    oh_ref[...] = p[0:N_NODES, 0:1]
    ov_ref[...] = p[0:N_NODES, 1:4]


def _emit_call(comb):
    return pl.pallas_call(
        _emit_body,
        out_shape=(
            jax.ShapeDtypeStruct((N_NODES, 1), jnp.float32),
            jax.ShapeDtypeStruct((N_NODES, 3), jnp.float32),
        ),
    )(comb)


def kernel(species, edge_index, edge_attr, edge_vec,
           W_sp, b_sp, W_e, b_e, W_msg, b_msg, W_vec, W_dh, b_dh, w_dv):
    f32 = jnp.float32
    hp = jax.lax.Precision.HIGHEST
    mm = functools.partial(jnp.matmul, precision=hp)
    wdh = W_dh[:, 0]
    wecat = jnp.concatenate([W_e, b_e[None, :]], axis=0)
    G_h = mm(W_msg, (wecat * wdh[None, :]).T)
    g_h = mm(wecat * wdh[None, :], b_msg)
    G_v = mm(W_vec, (wecat * w_dv[None, :]).T)
    M_p = mm(W_sp, G_h)
    c_p = mm(b_sp, G_h) + g_h
    M_q = mm(W_sp, G_v)
    c_q = mm(b_sp, G_v)
    m_b = mm(W_sp, wdh)
    c_b = jnp.dot(b_sp, wdh, precision=hp) + b_dh[0]

    M8 = jnp.zeros((8, 10), f32).at[0:5, 0:5].set(M_p).at[0:5, 5:10].set(M_q)
    c10 = jnp.concatenate([c_p, c_q])[None, :].astype(f32)
    mb8 = jnp.zeros((8, 1), f32).at[0:5, 0].set(m_b)
    cb = jnp.full((1, 1), c_b, f32)

    sp8 = jnp.zeros((NPAD, 8), f32).at[0:N_NODES, 0:5].set(species)

    npad_e = EPAD - N_EDGES
    src_p = jnp.concatenate(
        [edge_index[0], jnp.full((npad_e,), N_NODES, jnp.int32)])
    dst_p = jnp.concatenate(
        [edge_index[1], jnp.full((npad_e,), N_NODES, jnp.int32)])
    ea_p = jnp.concatenate(
        [edge_attr.T.astype(f32), jnp.zeros((4, npad_e), f32)], axis=1)

    table, base = _prep_call(sp8, M8, c10, mb8, cb)
    part = _sc_call(table.reshape(NPAD * 10), src_p, dst_p, ea_p)
    comb = _reduce_call(part, base.reshape(1, NPAD * 4))
    out_h, out_v = _emit_call(comb.reshape(NPAD, 4))
    return out_h, out_v


# speedup vs baseline: 124.5159x; 1.4352x over previous
import functools

import jax
import jax.numpy as jnp
from jax import lax
from jax.experimental import pallas as pl
from jax.experimental.pallas import tpu as pltpu
from jax.experimental.pallas import tpu_sc as plsc

N_NODES = 10000
N_EDGES = 160000
CUTOFF = 5.0

NC = 2
NS = 16
NW = NC * NS
NPAD = 10240
NH = NPAD // 2
EPAD = 163840
EPT = EPAD // NW
CH = 512
NCHUNK = EPT // CH
GRP = CH // 16

_PI = 3.14159265358979323846
_C3 = -1.0 / 6.0
_C5 = 1.0 / 120.0
_C7 = -1.0 / 5040.0
_C9 = 1.0 / 362880.0


def _prep_body(sp_ref, m_ref, c_ref, mb_ref, cb_ref, table_ref, base_ref):
    sp = sp_ref[...]
    table = jax.lax.dot_general(
        sp, m_ref[...], (((1,), (0,)), ((), ())),
        preferred_element_type=jnp.float32,
        precision=jax.lax.Precision.HIGHEST) + c_ref[...]
    base = jax.lax.dot_general(
        sp, mb_ref[...], (((1,), (0,)), ((), ())),
        preferred_element_type=jnp.float32,
        precision=jax.lax.Precision.HIGHEST) + cb_ref[...]
    rows10 = lax.broadcasted_iota(jnp.int32, (NPAD, 10), 0)
    table_ref[...] = jnp.where(rows10 < N_NODES, table, 0.0)
    rows4 = lax.broadcasted_iota(jnp.int32, (NPAD, 4), 0)
    cols4 = lax.broadcasted_iota(jnp.int32, (NPAD, 4), 1)
    base4 = jnp.broadcast_to(base, (NPAD, 4))
    base_ref[...] = jnp.where((rows4 < N_NODES) & (cols4 == 0), base4, 0.0)


def _prep_call(sp8, m8, c10, mb8, cb):
    return pl.pallas_call(
        _prep_body,
        out_shape=(
            jax.ShapeDtypeStruct((NPAD, 10), jnp.float32),
            jax.ShapeDtypeStruct((NPAD, 4), jnp.float32),
        ),
    )(sp8, m8, c10, mb8, cb)


def _sc_body(table_hbm, idx_hbm, ea_hbm, out_hbm,
             table_v, idx_v, ea_v, acc_v, *sems):
    cid = lax.axis_index("c")
    sid = lax.axis_index("s")
    wid = cid * NS + sid

    tbl_cp = pltpu.async_copy(table_hbm, table_v, sems[4])
    zeros16 = jnp.zeros((16,), jnp.float32)
    e0 = wid * EPT

    def issue(k, buf):
        off = pl.multiple_of(e0 + k * CH, CH)
        return (
            pltpu.async_copy(idx_hbm.at[:, pl.ds(off, CH)], idx_v.at[buf],
                             sems[buf]),
            pltpu.async_copy(ea_hbm.at[:, pl.ds(off, CH)], ea_v.at[buf],
                             sems[2 + buf]),
        )

    pending = issue(0, 0)
    table_waited = False

    for half in range(2):
        rbase = half * NH

        def zero(i, carry):
            acc_v[pl.ds(i * 16, 16)] = zeros16
            return carry

        lax.fori_loop(0, (NH * 4) // 16, zero, 0, unroll=4)
        if not table_waited:
            tbl_cp.wait()
            table_waited = True

        for k in range(NCHUNK):
            buf = k % 2
            for cp in pending:
                cp.wait()
            nxt = k + 1 if k + 1 < NCHUNK else (0 if half == 0 else None)
            if nxt is not None:
                pending = issue(nxt, (buf + 1) % 2)

            def group(g, carry):
                s16 = idx_v[buf, 0, pl.ds(g * 16, 16)] * 10
                tv = [plsc.load_gather(table_v, [s16 + j])
                      for j in range(10)]
                ea0 = ea_v[buf, 0, pl.ds(g * 16, 16)]
                ea1 = ea_v[buf, 1, pl.ds(g * 16, 16)]
                ea2 = ea_v[buf, 2, pl.ds(g * 16, 16)]
                ln = ea_v[buf, 3, pl.ds(g * 16, 16)]
                x = jnp.minimum(ln * (1.0 / CUTOFF), 1.0)
                z = x * _PI - (0.5 * _PI)
                z2 = z * z
                poly = _C3 + z2 * (_C5 + z2 * (_C7 + z2 * _C9))
                env = 0.5 - 0.5 * (z + z * z2 * poly)
                inv = 1.0 / (ln + 1e-8)
                dotp = ea0 * tv[0] + ea1 * tv[1] + ea2 * tv[2] \
                    + ln * tv[3] + tv[4]
                dotq = ea0 * tv[5] + ea1 * tv[6] + ea2 * tv[7] \
                    + ln * tv[8] + tv[9]
                t = env * dotp
                sv = env * dotq * inv
                d16 = idx_v[buf, 1, pl.ds(g * 16, 16)] - rbase
                m = (d16 >= 0) & (d16 < NH)
                ix = jnp.where(m, d16, 0) * 4
                plsc.addupdate_scatter(acc_v, [ix], t, mask=m)
                plsc.addupdate_scatter(acc_v, [ix + 1], sv * ea0, mask=m)
                plsc.addupdate_scatter(acc_v, [ix + 2], sv * ea1, mask=m)
                plsc.addupdate_scatter(acc_v, [ix + 3], sv * ea2, mask=m)
                return carry

            lax.fori_loop(0, GRP, group, 0, unroll=2)

        rb = pl.multiple_of(rbase * 4, NH * 4)
        pltpu.sync_copy(acc_v, out_hbm.at[wid, pl.ds(rb, NH * 4)])


def _sc_call(table, idx2, ea_p):
    mesh = plsc.VectorSubcoreMesh(core_axis_name="c", subcore_axis_name="s",
                                  num_cores=NC, num_subcores=NS)
    f = functools.partial(
        pl.kernel,
        out_type=jax.ShapeDtypeStruct((NW, NPAD * 4), jnp.float32),
        mesh=mesh,
        compiler_params=pltpu.CompilerParams(needs_layout_passes=False,
                                             use_tc_tiling_on_sc=False),
        scratch_types=[
            pltpu.VMEM((NPAD * 10,), jnp.float32),
            pltpu.VMEM((2, 2, CH), jnp.int32),
            pltpu.VMEM((2, 4, CH), jnp.float32),
            pltpu.VMEM((NH * 4,), jnp.float32),
            pltpu.SemaphoreType.DMA,
            pltpu.SemaphoreType.DMA,
            pltpu.SemaphoreType.DMA,
            pltpu.SemaphoreType.DMA,
            pltpu.SemaphoreType.DMA,
        ],
    )(_sc_body)
    return f(table, idx2, ea_p)


def _reduce_body(part_ref, base_ref, out_ref):
    out_ref[...] = jnp.sum(part_ref[...], axis=0, keepdims=True) + base_ref[...]


def _reduce_call(part2d, base2d):
    return pl.pallas_call(
        _reduce_body,
        out_shape=jax.ShapeDtypeStruct((1, NPAD * 4), jnp.float32),
    )(part2d, base2d)


def _emit_body(comb_ref, oh_ref, ov_ref):
    p = comb_ref[...]
    oh_ref[...] = p[0:N_NODES, 0:1]
    ov_ref[...] = p[0:N_NODES, 1:4]


def _emit_call(comb):
    return pl.pallas_call(
        _emit_body,
        out_shape=(
            jax.ShapeDtypeStruct((N_NODES, 1), jnp.float32),
            jax.ShapeDtypeStruct((N_NODES, 3), jnp.float32),
        ),
    )(comb)


def kernel(species, edge_index, edge_attr, edge_vec,
           W_sp, b_sp, W_e, b_e, W_msg, b_msg, W_vec, W_dh, b_dh, w_dv):
    f32 = jnp.float32
    hp = jax.lax.Precision.HIGHEST
    mm = functools.partial(jnp.matmul, precision=hp)
    wdh = W_dh[:, 0]
    wecat = jnp.concatenate([W_e, b_e[None, :]], axis=0)
    G_h = mm(W_msg, (wecat * wdh[None, :]).T)
    g_h = mm(wecat * wdh[None, :], b_msg)
    G_v = mm(W_vec, (wecat * w_dv[None, :]).T)
    M_p = mm(W_sp, G_h)
    c_p = mm(b_sp, G_h) + g_h
    M_q = mm(W_sp, G_v)
    c_q = mm(b_sp, G_v)
    m_b = mm(W_sp, wdh)
    c_b = jnp.dot(b_sp, wdh, precision=hp) + b_dh[0]

    M8 = jnp.zeros((8, 10), f32).at[0:5, 0:5].set(M_p).at[0:5, 5:10].set(M_q)
    c10 = jnp.concatenate([c_p, c_q])[None, :].astype(f32)
    mb8 = jnp.zeros((8, 1), f32).at[0:5, 0].set(m_b)
    cb = jnp.full((1, 1), c_b, f32)

    sp8 = jnp.zeros((NPAD, 8), f32).at[0:N_NODES, 0:5].set(species)

    npad_e = EPAD - N_EDGES
    idx2 = jnp.concatenate(
        [edge_index, jnp.full((2, npad_e), N_NODES, jnp.int32)], axis=1)
    ea_p = jnp.concatenate(
        [edge_attr.T.astype(f32), jnp.zeros((4, npad_e), f32)], axis=1)

    table, base = _prep_call(sp8, M8, c10, mb8, cb)
    part = _sc_call(table.reshape(NPAD * 10), idx2, ea_p)
    comb = _reduce_call(part, base.reshape(1, NPAD * 4))
    out_h, out_v = _emit_call(comb.reshape(NPAD, 4))
    return out_h, out_v
